# baseline (speedup 1.0000x reference)
import functools

import numpy as np

import jax
import jax.numpy as jnp
from jax import lax
from jax.experimental import pallas as pl
from jax.experimental.pallas import tpu as pltpu
from jax.experimental.pallas import tpu_sc as plsc

N = 10000
E = 320000
D = 128
H = 4
U = 32

NC = 2
NS = 16
HD = D // NC
EPT = E // NS
K = 80
CHUNKS = EPT // K
ACCW = 96
RPT = N // NS
ZR = 125

TCB = 2000


def _tc1_body(x_ref, w_ref, b_ref, scale_ref, shift_ref, p_ref, qw_ref, mlp_ref):
    y = jnp.dot(x_ref[:], w_ref[:], preferred_element_type=jnp.float32) + b_ref[:]
    w = y[:, 0:D]
    q = y[:, D:2 * D]
    p = y[:, 2 * D:3 * D]
    mlp = jnp.maximum(y[:, 3 * D:4 * D], 0.0) * scale_ref[:] + shift_ref[:]
    mlp_ref[:] = mlp
    for c in range(NC):
        p_ref[c] = p[:, HD * c:HD * c + HD].astype(jnp.bfloat16)
        qw_ref[c, :, 0:HD] = q[:, HD * c:HD * c + HD].astype(jnp.bfloat16)
        qw_ref[c, :, HD:2 * HD] = w[:, HD * c:HD * c + HD].astype(jnp.bfloat16)


def _tc2_body(acc_ref, mlp_ref, out_ref):
    for h in range(H):
        c, lh = divmod(h, NC)
        eh = acc_ref[c, :, 2 * U + 2 * lh:2 * U + 2 * lh + 1].astype(jnp.float32)
        msum = acc_ref[c, :, U * lh:U * lh + U].astype(jnp.float32)
        ph = jnp.maximum(msum / (eh + 1e-9), 0.0)
        out_ref[:, U * h:U * h + U] = mlp_ref[:, U * h:U * h + U] + ph


def _sc_body(idx_hbm, p_hbm, qw_hbm, wa2_hbm, out_hbm,
             idx_v, prows, qwrows, stage, zbuf, wa2_v,
             acc_sh, sem_p0, sem_p1, sem_q0, sem_q1, sem_s0, sem_s1):
    cid = lax.axis_index("c")
    sid = lax.axis_index("s")
    sem_p = [sem_p0, sem_p1]
    sem_q = [sem_q0, sem_q1]
    sem_s = [sem_s0, sem_s1]

    def zrow(r, carry):
        for s in range(ACCW // 32):
            zbuf[r, 32 * s:32 * s + 32] = jnp.zeros((32,), jnp.bfloat16)
        return carry

    lax.fori_loop(0, ZR, zrow, 0)
    for kk in range(RPT // ZR):
        pltpu.sync_copy(zbuf, acc_sh.at[pl.ds(sid * RPT + kk * ZR, ZR)])

    pltpu.sync_copy(wa2_hbm.at[cid], wa2_v)
    a_regs = [wa2_v[16 * s:16 * s + 16] for s in range(HD // 16)]
    lane = lax.iota(jnp.int32, 16)
    m0 = (lane == 0).astype(jnp.float32)
    m1 = (lane == 1).astype(jnp.float32)

    plsc.subcore_barrier()

    def load_idx(ci, b):
        pltpu.sync_copy(idx_hbm.at[sid * CHUNKS + ci], idx_v.at[b])

    def issue_gather(b):
        pltpu.async_copy(p_hbm.at[cid].at[idx_v.at[b, 0]], prows.at[b], sem_p[b])
        pltpu.async_copy(qw_hbm.at[cid].at[idx_v.at[b, 1]], qwrows.at[b], sem_q[b])

    load_idx(0, 0)
    issue_gather(0)

    def pair_body(g, carry):
        for b in (0, 1):
            ci = 2 * g + b
            nb = 1 - b
            pltpu.make_async_copy(
                p_hbm.at[cid].at[idx_v.at[b, 0]], prows.at[b], sem_p[b]).wait()
            pltpu.make_async_copy(
                qw_hbm.at[cid].at[idx_v.at[b, 1]], qwrows.at[b], sem_q[b]).wait()

            @pl.when(ci >= 1)
            def _():
                pltpu.make_async_copy(
                    stage.at[nb], acc_sh.at[idx_v.at[nb, 0]], sem_s[nb]).wait()

            @pl.when(ci + 1 < CHUNKS)
            def _():
                load_idx(ci + 1, nb)
                issue_gather(nb)

            @plsc.parallel_loop(0, K, unroll=2)
            def edge(j):
                def unpk(v):
                    return plsc.unpack(v, format=plsc.PackFormat.INTERLEAVED,
                                       preferred_element_type=jnp.float32)

                ps, qs, ws = [], [], []
                for g in range(HD // 32):
                    ps.extend(unpk(prows[b, j, 32 * g:32 * g + 32]))
                    qs.extend(unpk(qwrows[b, j, 32 * g:32 * g + 32]))
                    ws.extend(unpk(qwrows[b, j, HD + 32 * g:HD + 32 * g + 32]))
                zs = [jnp.maximum(ps[s] + qs[s], 0.0) for s in range(HD // 16)]
                ts = [zs[s] * a_regs[s] for s in range(HD // 16)]
                hs = [jnp.sum(ts[2 * lh] + ts[2 * lh + 1]) for lh in range(NC)]
                es = [jnp.exp(jnp.full((16,), hs[lh], jnp.float32))
                      for lh in range(NC)]
                for g in range(HD // 32):
                    stage[b, j, 32 * g:32 * g + 32] = plsc.pack(
                        ws[2 * g] * es[g], ws[2 * g + 1] * es[g],
                        format=plsc.PackFormat.INTERLEAVED)
                stage[b, j, HD:HD + 32] = plsc.pack(
                    es[0] * m0 + es[1] * m1, jnp.zeros((16,), jnp.float32),
                    format=plsc.PackFormat.INTERLEAVED)

            pltpu.async_copy(stage.at[b], acc_sh.at[idx_v.at[b, 0]], sem_s[b],
                             add=True)
        return carry

    lax.fori_loop(0, CHUNKS // 2, pair_body, 0)

    lb = (CHUNKS - 1) % 2
    pltpu.make_async_copy(
        stage.at[lb], acc_sh.at[idx_v.at[lb, 0]], sem_s[lb]).wait()

    plsc.subcore_barrier()

    for kk in range(RPT // ZR):
        rs = sid * RPT + kk * ZR
        pltpu.sync_copy(acc_sh.at[pl.ds(rs, ZR)], out_hbm.at[cid, pl.ds(rs, ZR)])


def kernel(node_attributes, edge_indices, Wt, bt, Wa, ba, wa2, Wm, bm,
           gamma, beta, moving_mean, moving_var):
    x = node_attributes
    recv = edge_indices[:, 0]
    send = edge_indices[:, 1]

    Ar = Wa[:, :U, :]
    As = Wa[:, U:2 * U, :]
    ae = Wa[:, 2 * U, :]
    Wtr = jnp.transpose(Wt, (1, 0, 2))
    Mw = Wtr.reshape(D, H * U)
    bw = bt.reshape(H * U)
    Mp = jnp.einsum('dhu,huv->dhv', Wtr, Ar).reshape(D, H * U)
    bp = (jnp.einsum('hu,huv->hv', bt, Ar) + ae + ba).reshape(H * U)
    Mq = jnp.einsum('dhu,huv->dhv', Wtr, As).reshape(D, H * U)
    bq = jnp.einsum('hu,huv->hv', bt, As).reshape(H * U)
    wbig = jnp.concatenate([Mw, Mq, Mp, Wm], axis=1)
    bbig = jnp.concatenate([bw, bq, bp, bm]).reshape(1, 4 * D)
    scale = (gamma / jnp.sqrt(moving_var + 1e-3)).reshape(1, D)
    shift = (beta - moving_mean * (gamma / jnp.sqrt(moving_var + 1e-3))).reshape(1, D)
    unperm = np.empty(HD, np.int32)
    for g in range(2):
        for i in range(16):
            unperm[32 * g + i] = 32 * g + 2 * i
            unperm[32 * g + 16 + i] = 32 * g + 2 * i + 1
    wa2v = wa2.reshape(NC, HD)[:, unperm]

    grid = N // TCB
    p_tab, qw_tab, mlp_tab = pl.pallas_call(
        _tc1_body,
        grid=(grid,),
        in_specs=[
            pl.BlockSpec((TCB, D), lambda i: (i, 0)),
            pl.BlockSpec((D, 4 * D), lambda i: (0, 0)),
            pl.BlockSpec((1, 4 * D), lambda i: (0, 0)),
            pl.BlockSpec((1, D), lambda i: (0, 0)),
            pl.BlockSpec((1, D), lambda i: (0, 0)),
        ],
        out_specs=[
            pl.BlockSpec((NC, TCB, HD), lambda i: (0, i, 0)),
            pl.BlockSpec((NC, TCB, 2 * HD), lambda i: (0, i, 0)),
            pl.BlockSpec((TCB, D), lambda i: (i, 0)),
        ],
        out_shape=[
            jax.ShapeDtypeStruct((NC, N, HD), jnp.bfloat16),
            jax.ShapeDtypeStruct((NC, N, 2 * HD), jnp.bfloat16),
            jax.ShapeDtypeStruct((N, D), jnp.float32),
        ],
    )(x, wbig, bbig, scale, shift)

    sc_kernel = functools.partial(
        pl.kernel,
        out_type=jax.ShapeDtypeStruct((NC, N, ACCW), jnp.bfloat16),
        mesh=plsc.VectorSubcoreMesh(core_axis_name="c", subcore_axis_name="s"),
        compiler_params=pltpu.CompilerParams(use_tc_tiling_on_sc=False,
                                             needs_layout_passes=False),
        scratch_types=[
            pltpu.VMEM((2, 2, K), jnp.int32),
            pltpu.VMEM((2, K, HD), jnp.bfloat16),
            pltpu.VMEM((2, K, 2 * HD), jnp.bfloat16),
            pltpu.VMEM((2, K, ACCW), jnp.bfloat16),
            pltpu.VMEM((ZR, ACCW), jnp.bfloat16),
            pltpu.VMEM((HD,), jnp.float32),
            pltpu.VMEM_SHARED((N, ACCW), jnp.bfloat16),
            pltpu.SemaphoreType.DMA,
            pltpu.SemaphoreType.DMA,
            pltpu.SemaphoreType.DMA,
            pltpu.SemaphoreType.DMA,
            pltpu.SemaphoreType.DMA,
            pltpu.SemaphoreType.DMA,
        ],
    )(_sc_body)
    idx_packed = jnp.stack([recv.reshape(E // K, K),
                            send.reshape(E // K, K)], axis=1)
    acc = sc_kernel(idx_packed, p_tab, qw_tab, wa2v)

    out = pl.pallas_call(
        _tc2_body,
        grid=(grid,),
        in_specs=[
            pl.BlockSpec((NC, TCB, ACCW), lambda i: (0, i, 0)),
            pl.BlockSpec((TCB, D), lambda i: (i, 0)),
        ],
        out_specs=pl.BlockSpec((TCB, D), lambda i: (i, 0)),
        out_shape=jax.ShapeDtypeStruct((N, D), jnp.float32),
    )(acc, mlp_tab)
    return out


# speedup vs baseline: 1.1349x; 1.1349x over previous
import functools

import numpy as np

import jax
import jax.numpy as jnp
from jax import lax
from jax.experimental import pallas as pl
from jax.experimental.pallas import tpu as pltpu
from jax.experimental.pallas import tpu_sc as plsc

N = 10000
E = 320000
D = 128
H = 4
U = 32

NC = 2
NS = 16
HD = D // NC
EPT = E // NS
K = 160
CHUNKS = EPT // K
ACCW = 96
RPT = N // NS
ZR = 125

TCB = 2000


def _tc1_body(x_ref, w_ref, b_ref, scale_ref, shift_ref, p_ref, qw_ref, mlp_ref):
    y = jnp.dot(x_ref[:], w_ref[:], preferred_element_type=jnp.float32) + b_ref[:]
    w = y[:, 0:D]
    q = y[:, D:2 * D]
    p = y[:, 2 * D:3 * D]
    mlp = jnp.maximum(y[:, 3 * D:4 * D], 0.0) * scale_ref[:] + shift_ref[:]
    mlp_ref[:] = mlp
    for c in range(NC):
        p_ref[c] = p[:, HD * c:HD * c + HD].astype(jnp.bfloat16)
        qw_ref[c, :, 0:HD] = q[:, HD * c:HD * c + HD].astype(jnp.bfloat16)
        qw_ref[c, :, HD:2 * HD] = w[:, HD * c:HD * c + HD].astype(jnp.bfloat16)


def _tc2_body(acc_ref, mlp_ref, out_ref):
    for h in range(H):
        c, lh = divmod(h, NC)
        eh = acc_ref[c, :, 2 * U + 2 * lh:2 * U + 2 * lh + 1].astype(jnp.float32)
        msum = acc_ref[c, :, U * lh:U * lh + U].astype(jnp.float32)
        ph = jnp.maximum(msum / (eh + 1e-9), 0.0)
        out_ref[:, U * h:U * h + U] = mlp_ref[:, U * h:U * h + U] + ph


def _sc_body(idx_hbm, p_hbm, qw_hbm, wa2_hbm, out_hbm,
             idx_v, prows, qwrows, stage, zbuf, wa2_v,
             acc_sh, sem_p0, sem_p1, sem_q0, sem_q1, sem_s0, sem_s1):
    cid = lax.axis_index("c")
    sid = lax.axis_index("s")
    sem_p = [sem_p0, sem_p1]
    sem_q = [sem_q0, sem_q1]
    sem_s = [sem_s0, sem_s1]

    def zrow(r, carry):
        for s in range(ACCW // 32):
            zbuf[r, 32 * s:32 * s + 32] = jnp.zeros((32,), jnp.bfloat16)
        return carry

    lax.fori_loop(0, ZR, zrow, 0)
    for kk in range(RPT // ZR):
        pltpu.sync_copy(zbuf, acc_sh.at[pl.ds(sid * RPT + kk * ZR, ZR)])

    pltpu.sync_copy(wa2_hbm.at[cid], wa2_v)
    a_regs = [wa2_v[16 * s:16 * s + 16] for s in range(HD // 16)]
    lane = lax.iota(jnp.int32, 16)
    m0 = (lane == 0).astype(jnp.float32)
    m1 = (lane == 1).astype(jnp.float32)

    plsc.subcore_barrier()

    def load_idx(ci, b):
        pltpu.sync_copy(idx_hbm.at[sid * CHUNKS + ci], idx_v.at[b])

    def issue_gather(b):
        pltpu.async_copy(p_hbm.at[cid].at[idx_v.at[b, 0]], prows.at[b], sem_p[b])
        pltpu.async_copy(qw_hbm.at[cid].at[idx_v.at[b, 1]], qwrows.at[b], sem_q[b])

    load_idx(0, 0)
    issue_gather(0)

    def compute_chunk(b):
        @plsc.parallel_loop(0, K, unroll=2)
        def edge(j):
            def unpk(v):
                return plsc.unpack(v, format=plsc.PackFormat.INTERLEAVED,
                                   preferred_element_type=jnp.float32)

            ps, qs, ws = [], [], []
            for g in range(HD // 32):
                ps.extend(unpk(prows[b, j, 32 * g:32 * g + 32]))
                qs.extend(unpk(qwrows[b, j, 32 * g:32 * g + 32]))
                ws.extend(unpk(qwrows[b, j, HD + 32 * g:HD + 32 * g + 32]))
            zs = [jnp.maximum(ps[s] + qs[s], 0.0) for s in range(HD // 16)]
            ts = [zs[s] * a_regs[s] for s in range(HD // 16)]
            hs = [jnp.sum(ts[2 * lh] + ts[2 * lh + 1]) for lh in range(NC)]
            es = [jnp.exp(jnp.full((16,), hs[lh], jnp.float32))
                  for lh in range(NC)]
            for g in range(HD // 32):
                stage[b, j, 32 * g:32 * g + 32] = plsc.pack(
                    ws[2 * g] * es[g], ws[2 * g + 1] * es[g],
                    format=plsc.PackFormat.INTERLEAVED)
            stage[b, j, HD:HD + 32] = plsc.pack(
                es[0] * m0 + es[1] * m1, jnp.zeros((16,), jnp.float32),
                format=plsc.PackFormat.INTERLEAVED)

    def wait_gather(b):
        pltpu.make_async_copy(
            p_hbm.at[cid].at[idx_v.at[b, 0]], prows.at[b], sem_p[b]).wait()
        pltpu.make_async_copy(
            qw_hbm.at[cid].at[idx_v.at[b, 1]], qwrows.at[b], sem_q[b]).wait()

    def wait_scatter(b):
        pltpu.make_async_copy(
            stage.at[b], acc_sh.at[idx_v.at[b, 0]], sem_s[b]).wait()

    def issue_scatter(b):
        pltpu.async_copy(stage.at[b], acc_sh.at[idx_v.at[b, 0]], sem_s[b],
                         add=True)

    def pair_body(g, carry):
        for b in (0, 1):
            ci = 2 * g + b
            nb = 1 - b
            wait_gather(b)

            @pl.when(ci >= 1)
            def _():
                wait_scatter(nb)

            @pl.when(ci + 1 < CHUNKS)
            def _():
                load_idx(ci + 1, nb)
                issue_gather(nb)

            compute_chunk(b)
            issue_scatter(b)
        return carry

    lax.fori_loop(0, CHUNKS // 2, pair_body, 0)

    if CHUNKS % 2 == 1:
        wait_gather(0)
        wait_scatter(1)
        compute_chunk(0)
        issue_scatter(0)
        wait_scatter(0)
    else:
        wait_scatter((CHUNKS - 1) % 2)

    plsc.subcore_barrier()

    for kk in range(RPT // ZR):
        rs = sid * RPT + kk * ZR
        pltpu.sync_copy(acc_sh.at[pl.ds(rs, ZR)], out_hbm.at[cid, pl.ds(rs, ZR)])


def kernel(node_attributes, edge_indices, Wt, bt, Wa, ba, wa2, Wm, bm,
           gamma, beta, moving_mean, moving_var):
    x = node_attributes
    recv = edge_indices[:, 0]
    send = edge_indices[:, 1]

    Ar = Wa[:, :U, :]
    As = Wa[:, U:2 * U, :]
    ae = Wa[:, 2 * U, :]
    Wtr = jnp.transpose(Wt, (1, 0, 2))
    Mw = Wtr.reshape(D, H * U)
    bw = bt.reshape(H * U)
    Mp = jnp.einsum('dhu,huv->dhv', Wtr, Ar).reshape(D, H * U)
    bp = (jnp.einsum('hu,huv->hv', bt, Ar) + ae + ba).reshape(H * U)
    Mq = jnp.einsum('dhu,huv->dhv', Wtr, As).reshape(D, H * U)
    bq = jnp.einsum('hu,huv->hv', bt, As).reshape(H * U)
    wbig = jnp.concatenate([Mw, Mq, Mp, Wm], axis=1)
    bbig = jnp.concatenate([bw, bq, bp, bm]).reshape(1, 4 * D)
    scale = (gamma / jnp.sqrt(moving_var + 1e-3)).reshape(1, D)
    shift = (beta - moving_mean * (gamma / jnp.sqrt(moving_var + 1e-3))).reshape(1, D)
    unperm = np.empty(HD, np.int32)
    for g in range(2):
        for i in range(16):
            unperm[32 * g + i] = 32 * g + 2 * i
            unperm[32 * g + 16 + i] = 32 * g + 2 * i + 1
    wa2v = wa2.reshape(NC, HD)[:, unperm]

    grid = N // TCB
    p_tab, qw_tab, mlp_tab = pl.pallas_call(
        _tc1_body,
        grid=(grid,),
        in_specs=[
            pl.BlockSpec((TCB, D), lambda i: (i, 0)),
            pl.BlockSpec((D, 4 * D), lambda i: (0, 0)),
            pl.BlockSpec((1, 4 * D), lambda i: (0, 0)),
            pl.BlockSpec((1, D), lambda i: (0, 0)),
            pl.BlockSpec((1, D), lambda i: (0, 0)),
        ],
        out_specs=[
            pl.BlockSpec((NC, TCB, HD), lambda i: (0, i, 0)),
            pl.BlockSpec((NC, TCB, 2 * HD), lambda i: (0, i, 0)),
            pl.BlockSpec((TCB, D), lambda i: (i, 0)),
        ],
        out_shape=[
            jax.ShapeDtypeStruct((NC, N, HD), jnp.bfloat16),
            jax.ShapeDtypeStruct((NC, N, 2 * HD), jnp.bfloat16),
            jax.ShapeDtypeStruct((N, D), jnp.float32),
        ],
    )(x, wbig, bbig, scale, shift)

    sc_kernel = functools.partial(
        pl.kernel,
        out_type=jax.ShapeDtypeStruct((NC, N, ACCW), jnp.bfloat16),
        mesh=plsc.VectorSubcoreMesh(core_axis_name="c", subcore_axis_name="s"),
        compiler_params=pltpu.CompilerParams(use_tc_tiling_on_sc=False,
                                             needs_layout_passes=False),
        scratch_types=[
            pltpu.VMEM((2, 2, K), jnp.int32),
            pltpu.VMEM((2, K, HD), jnp.bfloat16),
            pltpu.VMEM((2, K, 2 * HD), jnp.bfloat16),
            pltpu.VMEM((2, K, ACCW), jnp.bfloat16),
            pltpu.VMEM((ZR, ACCW), jnp.bfloat16),
            pltpu.VMEM((HD,), jnp.float32),
            pltpu.VMEM_SHARED((N, ACCW), jnp.bfloat16),
            pltpu.SemaphoreType.DMA,
            pltpu.SemaphoreType.DMA,
            pltpu.SemaphoreType.DMA,
            pltpu.SemaphoreType.DMA,
            pltpu.SemaphoreType.DMA,
            pltpu.SemaphoreType.DMA,
        ],
    )(_sc_body)
    idx_packed = jnp.stack([recv.reshape(E // K, K),
                            send.reshape(E // K, K)], axis=1)
    acc = sc_kernel(idx_packed, p_tab, qw_tab, wa2v)

    out = pl.pallas_call(
        _tc2_body,
        grid=(grid,),
        in_specs=[
            pl.BlockSpec((NC, TCB, ACCW), lambda i: (0, i, 0)),
            pl.BlockSpec((TCB, D), lambda i: (i, 0)),
        ],
        out_specs=pl.BlockSpec((TCB, D), lambda i: (i, 0)),
        out_shape=jax.ShapeDtypeStruct((N, D), jnp.float32),
    )(acc, mlp_tab)
    return out


# speedup vs baseline: 1.3264x; 1.1688x over previous
import functools

import numpy as np

import jax
import jax.numpy as jnp
from jax import lax
from jax.experimental import pallas as pl
from jax.experimental.pallas import tpu as pltpu
from jax.experimental.pallas import tpu_sc as plsc

N = 10000
E = 320000
D = 128
H = 4
U = 32

NC = 2
NS = 16
HD = D // NC
EPT = E // NS
K = 160
CHUNKS = EPT // K
ACCW = 96
RPT = N // NS
ZR = 125

TCB = 2000


def _tc1_body(x_ref, w_ref, b_ref, scale_ref, shift_ref, p_ref, qw_ref, mlp_ref):
    y = jnp.dot(x_ref[:], w_ref[:], preferred_element_type=jnp.float32) + b_ref[:]
    w = y[:, 0:D]
    q = y[:, D:2 * D]
    p = y[:, 2 * D:3 * D]
    mlp = jnp.maximum(y[:, 3 * D:4 * D], 0.0) * scale_ref[:] + shift_ref[:]
    mlp_ref[:] = mlp
    for c in range(NC):
        p_ref[c] = p[:, HD * c:HD * c + HD].astype(jnp.bfloat16)
        qw_ref[c, :, 0:HD] = q[:, HD * c:HD * c + HD].astype(jnp.bfloat16)
        qw_ref[c, :, HD:2 * HD] = w[:, HD * c:HD * c + HD].astype(jnp.bfloat16)


def _tc2_body(acc_ref, mlp_ref, out_ref):
    for h in range(H):
        c, lh = divmod(h, NC)
        eh = acc_ref[c, :, 2 * U + 2 * lh:2 * U + 2 * lh + 1].astype(jnp.float32)
        msum = acc_ref[c, :, U * lh:U * lh + U].astype(jnp.float32)
        ph = jnp.maximum(msum / (eh + 1e-9), 0.0)
        out_ref[:, U * h:U * h + U] = mlp_ref[:, U * h:U * h + U] + ph


def _sc_body(idx_hbm, p_hbm, qw_hbm, wa2_hbm, out_hbm,
             idx_v, prows, qwrows, stage, zbuf, wa2_v,
             acc_sh, sem_p0, sem_p1, sem_q0, sem_q1, sem_s0, sem_s1):
    cid = lax.axis_index("c")
    sid = lax.axis_index("s")
    sem_p = [sem_p0, sem_p1]
    sem_q = [sem_q0, sem_q1]
    sem_s = [sem_s0, sem_s1]

    def zrow(r, carry):
        for s in range(ACCW // 32):
            zbuf[r, 32 * s:32 * s + 32] = jnp.zeros((32,), jnp.bfloat16)
        return carry

    lax.fori_loop(0, ZR, zrow, 0)
    for kk in range(RPT // ZR):
        pltpu.sync_copy(zbuf, acc_sh.at[pl.ds(sid * RPT + kk * ZR, ZR)])

    pltpu.sync_copy(wa2_hbm.at[cid], wa2_v)
    a_regs = [wa2_v[16 * s:16 * s + 16] for s in range(HD // 16)]
    lane = lax.iota(jnp.int32, 16)
    m0 = (lane == 0).astype(jnp.float32)
    m1 = (lane == 1).astype(jnp.float32)

    plsc.subcore_barrier()

    pltpu.sync_copy(idx_hbm.at[pl.ds(sid * CHUNKS, CHUNKS)], idx_v)

    def issue_gather(ci, b):
        pltpu.async_copy(p_hbm.at[cid].at[idx_v.at[ci, 0]], prows.at[b], sem_p[b])
        pltpu.async_copy(qw_hbm.at[cid].at[idx_v.at[ci, 1]], qwrows.at[b], sem_q[b])

    issue_gather(0, 0)

    def compute_chunk(b):
        @plsc.parallel_loop(0, K, unroll=2)
        def edge(j):
            def unpk(v):
                return plsc.unpack(v, format=plsc.PackFormat.INTERLEAVED,
                                   preferred_element_type=jnp.float32)

            ps, qs, ws = [], [], []
            for g in range(HD // 32):
                ps.extend(unpk(prows[b, j, 32 * g:32 * g + 32]))
                qs.extend(unpk(qwrows[b, j, 32 * g:32 * g + 32]))
                ws.extend(unpk(qwrows[b, j, HD + 32 * g:HD + 32 * g + 32]))
            zs = [jnp.maximum(ps[s] + qs[s], 0.0) for s in range(HD // 16)]
            ts = [zs[s] * a_regs[s] for s in range(HD // 16)]
            hs = [jnp.sum(ts[2 * lh] + ts[2 * lh + 1]) for lh in range(NC)]
            es = [jnp.exp(jnp.full((16,), hs[lh], jnp.float32))
                  for lh in range(NC)]
            for g in range(HD // 32):
                stage[b, j, 32 * g:32 * g + 32] = plsc.pack(
                    ws[2 * g] * es[g], ws[2 * g + 1] * es[g],
                    format=plsc.PackFormat.INTERLEAVED)
            stage[b, j, HD:HD + 32] = plsc.pack(
                es[0] * m0 + es[1] * m1, jnp.zeros((16,), jnp.float32),
                format=plsc.PackFormat.INTERLEAVED)

    def wait_gather(ci, b):
        pltpu.make_async_copy(
            p_hbm.at[cid].at[idx_v.at[ci, 0]], prows.at[b], sem_p[b]).wait()
        pltpu.make_async_copy(
            qw_hbm.at[cid].at[idx_v.at[ci, 1]], qwrows.at[b], sem_q[b]).wait()

    def wait_scatter(ci, b):
        pltpu.make_async_copy(
            stage.at[b], acc_sh.at[idx_v.at[ci, 0]], sem_s[b]).wait()

    def issue_scatter(ci, b):
        pltpu.async_copy(stage.at[b], acc_sh.at[idx_v.at[ci, 0]], sem_s[b],
                         add=True)

    def pair_body(g, carry):
        for b in (0, 1):
            ci = 2 * g + b
            nb = 1 - b
            wait_gather(ci, b)

            @pl.when(ci >= 1)
            def _():
                wait_scatter(ci - 1, nb)

            @pl.when(ci + 1 < CHUNKS)
            def _():
                issue_gather(ci + 1, nb)

            compute_chunk(b)
            issue_scatter(ci, b)
        return carry

    lax.fori_loop(0, CHUNKS // 2, pair_body, 0)

    if CHUNKS % 2 == 1:
        wait_gather(CHUNKS - 1, 0)
        wait_scatter(CHUNKS - 2, 1)
        compute_chunk(0)
        issue_scatter(CHUNKS - 1, 0)
        wait_scatter(CHUNKS - 1, 0)
    else:
        wait_scatter(CHUNKS - 1, (CHUNKS - 1) % 2)

    plsc.subcore_barrier()

    for kk in range(RPT // ZR):
        rs = sid * RPT + kk * ZR
        pltpu.sync_copy(acc_sh.at[pl.ds(rs, ZR)], out_hbm.at[cid, pl.ds(rs, ZR)])


def kernel(node_attributes, edge_indices, Wt, bt, Wa, ba, wa2, Wm, bm,
           gamma, beta, moving_mean, moving_var):
    x = node_attributes
    recv = edge_indices[:, 0]
    send = edge_indices[:, 1]

    Ar = Wa[:, :U, :]
    As = Wa[:, U:2 * U, :]
    ae = Wa[:, 2 * U, :]
    Wtr = jnp.transpose(Wt, (1, 0, 2))
    Mw = Wtr.reshape(D, H * U)
    bw = bt.reshape(H * U)
    Mp = jnp.einsum('dhu,huv->dhv', Wtr, Ar).reshape(D, H * U)
    bp = (jnp.einsum('hu,huv->hv', bt, Ar) + ae + ba).reshape(H * U)
    Mq = jnp.einsum('dhu,huv->dhv', Wtr, As).reshape(D, H * U)
    bq = jnp.einsum('hu,huv->hv', bt, As).reshape(H * U)
    wbig = jnp.concatenate([Mw, Mq, Mp, Wm], axis=1)
    bbig = jnp.concatenate([bw, bq, bp, bm]).reshape(1, 4 * D)
    scale = (gamma / jnp.sqrt(moving_var + 1e-3)).reshape(1, D)
    shift = (beta - moving_mean * (gamma / jnp.sqrt(moving_var + 1e-3))).reshape(1, D)
    unperm = np.empty(HD, np.int32)
    for g in range(2):
        for i in range(16):
            unperm[32 * g + i] = 32 * g + 2 * i
            unperm[32 * g + 16 + i] = 32 * g + 2 * i + 1
    wa2v = wa2.reshape(NC, HD)[:, unperm]

    grid = N // TCB
    p_tab, qw_tab, mlp_tab = pl.pallas_call(
        _tc1_body,
        grid=(grid,),
        in_specs=[
            pl.BlockSpec((TCB, D), lambda i: (i, 0)),
            pl.BlockSpec((D, 4 * D), lambda i: (0, 0)),
            pl.BlockSpec((1, 4 * D), lambda i: (0, 0)),
            pl.BlockSpec((1, D), lambda i: (0, 0)),
            pl.BlockSpec((1, D), lambda i: (0, 0)),
        ],
        out_specs=[
            pl.BlockSpec((NC, TCB, HD), lambda i: (0, i, 0)),
            pl.BlockSpec((NC, TCB, 2 * HD), lambda i: (0, i, 0)),
            pl.BlockSpec((TCB, D), lambda i: (i, 0)),
        ],
        out_shape=[
            jax.ShapeDtypeStruct((NC, N, HD), jnp.bfloat16),
            jax.ShapeDtypeStruct((NC, N, 2 * HD), jnp.bfloat16),
            jax.ShapeDtypeStruct((N, D), jnp.float32),
        ],
    )(x, wbig, bbig, scale, shift)

    sc_kernel = functools.partial(
        pl.kernel,
        out_type=jax.ShapeDtypeStruct((NC, N, ACCW), jnp.bfloat16),
        mesh=plsc.VectorSubcoreMesh(core_axis_name="c", subcore_axis_name="s"),
        compiler_params=pltpu.CompilerParams(use_tc_tiling_on_sc=False,
                                             needs_layout_passes=False),
        scratch_types=[
            pltpu.VMEM((CHUNKS, 2, K), jnp.int32),
            pltpu.VMEM((2, K, HD), jnp.bfloat16),
            pltpu.VMEM((2, K, 2 * HD), jnp.bfloat16),
            pltpu.VMEM((2, K, ACCW), jnp.bfloat16),
            pltpu.VMEM((ZR, ACCW), jnp.bfloat16),
            pltpu.VMEM((HD,), jnp.float32),
            pltpu.VMEM_SHARED((N, ACCW), jnp.bfloat16),
            pltpu.SemaphoreType.DMA,
            pltpu.SemaphoreType.DMA,
            pltpu.SemaphoreType.DMA,
            pltpu.SemaphoreType.DMA,
            pltpu.SemaphoreType.DMA,
            pltpu.SemaphoreType.DMA,
        ],
    )(_sc_body)
    idx_packed = jnp.stack([recv.reshape(E // K, K),
                            send.reshape(E // K, K)], axis=1)
    acc = sc_kernel(idx_packed, p_tab, qw_tab, wa2v)

    out = pl.pallas_call(
        _tc2_body,
        grid=(grid,),
        in_specs=[
            pl.BlockSpec((NC, TCB, ACCW), lambda i: (0, i, 0)),
            pl.BlockSpec((TCB, D), lambda i: (i, 0)),
        ],
        out_specs=pl.BlockSpec((TCB, D), lambda i: (i, 0)),
        out_shape=jax.ShapeDtypeStruct((N, D), jnp.float32),
    )(acc, mlp_tab)
    return out


# speedup vs baseline: 1.3472x; 1.0156x over previous
import functools

import numpy as np

import jax
import jax.numpy as jnp
from jax import lax
from jax.experimental import pallas as pl
from jax.experimental.pallas import tpu as pltpu
from jax.experimental.pallas import tpu_sc as plsc

N = 10000
E = 320000
D = 128
H = 4
U = 32

NC = 2
NS = 16
HD = D // NC
EPT = E // NS
K = 160
CHUNKS = EPT // K
ACCW = 96
RPT = N // NS
ZR = 125

TCB = 2000


def _tc1_body(x_ref, w_ref, b_ref, scale_ref, shift_ref, p_ref, qw_ref, mlp_ref):
    y = jnp.dot(x_ref[:], w_ref[:], preferred_element_type=jnp.float32) + b_ref[:]
    w = y[:, 0:D]
    q = y[:, D:2 * D]
    p = y[:, 2 * D:3 * D]
    mlp = jnp.maximum(y[:, 3 * D:4 * D], 0.0) * scale_ref[:] + shift_ref[:]
    mlp_ref[:] = mlp
    for c in range(NC):
        p_ref[c] = p[:, HD * c:HD * c + HD].astype(jnp.bfloat16)
        qw_ref[c, :, 0:HD] = q[:, HD * c:HD * c + HD].astype(jnp.bfloat16)
        qw_ref[c, :, HD:2 * HD] = w[:, HD * c:HD * c + HD].astype(jnp.bfloat16)


def _tc2_body(acc_ref, mlp_ref, out_ref):
    for h in range(H):
        c, lh = divmod(h, NC)
        eh = acc_ref[c, :, 2 * U + 2 * lh:2 * U + 2 * lh + 1].astype(jnp.float32)
        msum = acc_ref[c, :, U * lh:U * lh + U].astype(jnp.float32)
        ph = jnp.maximum(msum / (eh + 1e-9), 0.0)
        out_ref[:, U * h:U * h + U] = mlp_ref[:, U * h:U * h + U] + ph


def _sc_body(idx_hbm, p_hbm, qw_hbm, wa2_hbm, out_hbm,
             idx_v, prows, qwrows, stage, zbuf, wa2_v,
             acc_sh, sem_p0, sem_p1, sem_q0, sem_q1, sem_s0, sem_s1):
    cid = lax.axis_index("c")
    sid = lax.axis_index("s")
    sem_p = [sem_p0, sem_p1]
    sem_q = [sem_q0, sem_q1]
    sem_s = [sem_s0, sem_s1]

    def zrow(r, carry):
        for s in range(ACCW // 32):
            zbuf[r, 32 * s:32 * s + 32] = jnp.zeros((32,), jnp.bfloat16)
        return carry

    lax.fori_loop(0, ZR, zrow, 0)
    for kk in range(RPT // ZR):
        pltpu.sync_copy(zbuf, acc_sh.at[pl.ds(sid * RPT + kk * ZR, ZR)])

    pltpu.sync_copy(wa2_hbm.at[cid], wa2_v)
    a_regs = [wa2_v[16 * s:16 * s + 16] for s in range(HD // 16)]
    lane = lax.iota(jnp.int32, 16)
    m0 = (lane == 0).astype(jnp.float32)
    m1 = (lane == 1).astype(jnp.float32)

    plsc.subcore_barrier()

    pltpu.sync_copy(idx_hbm.at[pl.ds(sid * CHUNKS, CHUNKS)], idx_v)

    def issue_gather(ci, b):
        pltpu.async_copy(p_hbm.at[cid].at[idx_v.at[ci, 0]], prows.at[b], sem_p[b])
        pltpu.async_copy(qw_hbm.at[cid].at[idx_v.at[ci, 1]], qwrows.at[b], sem_q[b])

    issue_gather(0, 0)

    def compute_chunk(b):
        @plsc.parallel_loop(0, K, unroll=4)
        def edge(j):
            def unpk(v):
                return plsc.unpack(v, format=plsc.PackFormat.INTERLEAVED,
                                   preferred_element_type=jnp.float32)

            ps, qs, ws = [], [], []
            for g in range(HD // 32):
                ps.extend(unpk(prows[b, j, 32 * g:32 * g + 32]))
                qs.extend(unpk(qwrows[b, j, 32 * g:32 * g + 32]))
                ws.extend(unpk(qwrows[b, j, HD + 32 * g:HD + 32 * g + 32]))
            zs = [jnp.maximum(ps[s] + qs[s], 0.0) for s in range(HD // 16)]
            ts = [zs[s] * a_regs[s] for s in range(HD // 16)]
            hs = [jnp.sum(ts[2 * lh] + ts[2 * lh + 1]) for lh in range(NC)]
            es = [jnp.exp(jnp.full((16,), hs[lh], jnp.float32))
                  for lh in range(NC)]
            for g in range(HD // 32):
                stage[b, j, 32 * g:32 * g + 32] = plsc.pack(
                    ws[2 * g] * es[g], ws[2 * g + 1] * es[g],
                    format=plsc.PackFormat.INTERLEAVED)
            stage[b, j, HD:HD + 32] = plsc.pack(
                es[0] * m0 + es[1] * m1, jnp.zeros((16,), jnp.float32),
                format=plsc.PackFormat.INTERLEAVED)

    def wait_gather(ci, b):
        pltpu.make_async_copy(
            p_hbm.at[cid].at[idx_v.at[ci, 0]], prows.at[b], sem_p[b]).wait()
        pltpu.make_async_copy(
            qw_hbm.at[cid].at[idx_v.at[ci, 1]], qwrows.at[b], sem_q[b]).wait()

    def wait_scatter(ci, b):
        pltpu.make_async_copy(
            stage.at[b], acc_sh.at[idx_v.at[ci, 0]], sem_s[b]).wait()

    def issue_scatter(ci, b):
        pltpu.async_copy(stage.at[b], acc_sh.at[idx_v.at[ci, 0]], sem_s[b],
                         add=True)

    def pair_body(g, carry):
        for b in (0, 1):
            ci = 2 * g + b
            nb = 1 - b
            wait_gather(ci, b)

            @pl.when(ci >= 1)
            def _():
                wait_scatter(ci - 1, nb)

            @pl.when(ci + 1 < CHUNKS)
            def _():
                issue_gather(ci + 1, nb)

            compute_chunk(b)
            issue_scatter(ci, b)
        return carry

    lax.fori_loop(0, CHUNKS // 2, pair_body, 0)

    if CHUNKS % 2 == 1:
        wait_gather(CHUNKS - 1, 0)
        wait_scatter(CHUNKS - 2, 1)
        compute_chunk(0)
        issue_scatter(CHUNKS - 1, 0)
        wait_scatter(CHUNKS - 1, 0)
    else:
        wait_scatter(CHUNKS - 1, (CHUNKS - 1) % 2)

    plsc.subcore_barrier()

    for kk in range(RPT // ZR):
        rs = sid * RPT + kk * ZR
        pltpu.sync_copy(acc_sh.at[pl.ds(rs, ZR)], out_hbm.at[cid, pl.ds(rs, ZR)])


def kernel(node_attributes, edge_indices, Wt, bt, Wa, ba, wa2, Wm, bm,
           gamma, beta, moving_mean, moving_var):
    x = node_attributes
    recv = edge_indices[:, 0]
    send = edge_indices[:, 1]

    Ar = Wa[:, :U, :]
    As = Wa[:, U:2 * U, :]
    ae = Wa[:, 2 * U, :]
    Wtr = jnp.transpose(Wt, (1, 0, 2))
    Mw = Wtr.reshape(D, H * U)
    bw = bt.reshape(H * U)
    Mp = jnp.einsum('dhu,huv->dhv', Wtr, Ar).reshape(D, H * U)
    bp = (jnp.einsum('hu,huv->hv', bt, Ar) + ae + ba).reshape(H * U)
    Mq = jnp.einsum('dhu,huv->dhv', Wtr, As).reshape(D, H * U)
    bq = jnp.einsum('hu,huv->hv', bt, As).reshape(H * U)
    wbig = jnp.concatenate([Mw, Mq, Mp, Wm], axis=1)
    bbig = jnp.concatenate([bw, bq, bp, bm]).reshape(1, 4 * D)
    scale = (gamma / jnp.sqrt(moving_var + 1e-3)).reshape(1, D)
    shift = (beta - moving_mean * (gamma / jnp.sqrt(moving_var + 1e-3))).reshape(1, D)
    unperm = np.empty(HD, np.int32)
    for g in range(2):
        for i in range(16):
            unperm[32 * g + i] = 32 * g + 2 * i
            unperm[32 * g + 16 + i] = 32 * g + 2 * i + 1
    wa2v = wa2.reshape(NC, HD)[:, unperm]

    grid = N // TCB
    p_tab, qw_tab, mlp_tab = pl.pallas_call(
        _tc1_body,
        grid=(grid,),
        in_specs=[
            pl.BlockSpec((TCB, D), lambda i: (i, 0)),
            pl.BlockSpec((D, 4 * D), lambda i: (0, 0)),
            pl.BlockSpec((1, 4 * D), lambda i: (0, 0)),
            pl.BlockSpec((1, D), lambda i: (0, 0)),
            pl.BlockSpec((1, D), lambda i: (0, 0)),
        ],
        out_specs=[
            pl.BlockSpec((NC, TCB, HD), lambda i: (0, i, 0)),
            pl.BlockSpec((NC, TCB, 2 * HD), lambda i: (0, i, 0)),
            pl.BlockSpec((TCB, D), lambda i: (i, 0)),
        ],
        out_shape=[
            jax.ShapeDtypeStruct((NC, N, HD), jnp.bfloat16),
            jax.ShapeDtypeStruct((NC, N, 2 * HD), jnp.bfloat16),
            jax.ShapeDtypeStruct((N, D), jnp.float32),
        ],
    )(x, wbig, bbig, scale, shift)

    sc_kernel = functools.partial(
        pl.kernel,
        out_type=jax.ShapeDtypeStruct((NC, N, ACCW), jnp.bfloat16),
        mesh=plsc.VectorSubcoreMesh(core_axis_name="c", subcore_axis_name="s"),
        compiler_params=pltpu.CompilerParams(use_tc_tiling_on_sc=False,
                                             needs_layout_passes=False),
        scratch_types=[
            pltpu.VMEM((CHUNKS, 2, K), jnp.int32),
            pltpu.VMEM((2, K, HD), jnp.bfloat16),
            pltpu.VMEM((2, K, 2 * HD), jnp.bfloat16),
            pltpu.VMEM((2, K, ACCW), jnp.bfloat16),
            pltpu.VMEM((ZR, ACCW), jnp.bfloat16),
            pltpu.VMEM((HD,), jnp.float32),
            pltpu.VMEM_SHARED((N, ACCW), jnp.bfloat16),
            pltpu.SemaphoreType.DMA,
            pltpu.SemaphoreType.DMA,
            pltpu.SemaphoreType.DMA,
            pltpu.SemaphoreType.DMA,
            pltpu.SemaphoreType.DMA,
            pltpu.SemaphoreType.DMA,
        ],
    )(_sc_body)
    idx_packed = jnp.stack([recv.reshape(E // K, K),
                            send.reshape(E // K, K)], axis=1)
    acc = sc_kernel(idx_packed, p_tab, qw_tab, wa2v)

    out = pl.pallas_call(
        _tc2_body,
        grid=(grid,),
        in_specs=[
            pl.BlockSpec((NC, TCB, ACCW), lambda i: (0, i, 0)),
            pl.BlockSpec((TCB, D), lambda i: (i, 0)),
        ],
        out_specs=pl.BlockSpec((TCB, D), lambda i: (i, 0)),
        out_shape=jax.ShapeDtypeStruct((N, D), jnp.float32),
    )(acc, mlp_tab)
    return out
